# baseline (speedup 1.0000x reference)
import functools

import jax
import jax.numpy as jnp
from jax import lax
from jax.experimental import pallas as pl
from jax.experimental.pallas import tpu as pltpu
from jax.experimental.pallas import tpu_sc as plsc

W = 4096
F = 64
D = 8
K = 2 * D + 1
B = 8

NSC = 3

NUM_WORKERS = 32
ROWS_PER_WORKER = (NSC * W) // NUM_WORKERS
CH = 128
NCHUNK = ROWS_PER_WORKER // CH
NBUF = 2

DELAYS = [0] + [i for i in range(1, D + 1)] + [-i for i in range(1, D + 1)]

LANES = 16
FGROUPS = F // LANES


def _sc_body(subed_hbm, sub_hbm, out_hbm, sub_bufs, subed_bufs, out_bufs,
             sems_in, sems_out):
    wid = lax.axis_index("s") * 2 + lax.axis_index("c")
    row0 = wid * ROWS_PER_WORKER

    def chunk_coords(c):
        g = row0 + c * CH
        return g // W, g % W

    def sub_copy_mid(c, p):
        b, wloc = chunk_coords(c)
        return pltpu.make_async_copy(
            sub_hbm.at[b, pl.ds(wloc - D, CH + 2 * D)], sub_bufs[p],
            sems_in.at[p, 0])

    def sub_copy_first(c, p):
        b, _ = chunk_coords(c)
        return pltpu.make_async_copy(
            sub_hbm.at[b, pl.ds(0, CH + D)],
            sub_bufs[p].at[pl.ds(D, CH + D)], sems_in.at[p, 0])

    def sub_copy_last(c, p):
        b, wloc = chunk_coords(c)
        return pltpu.make_async_copy(
            sub_hbm.at[b, pl.ds(wloc - D, CH + D)],
            sub_bufs[p].at[pl.ds(0, CH + D)], sems_in.at[p, 0])

    def subed_copy(c, p):
        b, wloc = chunk_coords(c)
        return pltpu.make_async_copy(
            subed_hbm.at[b, pl.ds(wloc, CH)], subed_bufs[p],
            sems_in.at[p, 1])

    def edge_preds(c):
        _, wloc = chunk_coords(c)
        return wloc == 0, wloc == W - CH

    def start_in(c, p):
        subed_copy(c, p).start()
        is_first, is_last = edge_preds(c)

        @pl.when(is_first)
        def _():
            sub_copy_first(c, p).start()

        @pl.when(is_last)
        def _():
            sub_copy_last(c, p).start()

        @pl.when(jnp.logical_not(jnp.logical_or(is_first, is_last)))
        def _():
            sub_copy_mid(c, p).start()

    def wait_in(c, p):
        subed_copy(c, p).wait()
        is_first, is_last = edge_preds(c)

        @pl.when(is_first)
        def _():
            sub_copy_first(c, p).wait()
            for f in range(FGROUPS):
                fs = pl.ds(f * LANES, LANES)
                v = sub_bufs[p][D, fs]
                for t in range(D):
                    sub_bufs[p][t, fs] = v

        @pl.when(is_last)
        def _():
            sub_copy_last(c, p).wait()
            for f in range(FGROUPS):
                fs = pl.ds(f * LANES, LANES)
                v = sub_bufs[p][CH + D - 1, fs]
                for t in range(CH + D, CH + 2 * D):
                    sub_bufs[p][t, fs] = v

        @pl.when(jnp.logical_not(jnp.logical_or(is_first, is_last)))
        def _():
            sub_copy_mid(c, p).wait()

    def out_copy(c, p):
        b, wloc = chunk_coords(c)
        return pltpu.make_async_copy(
            out_bufs[p], out_hbm.at[b, pl.ds(wloc, CH)], sems_out.at[p])

    for c in range(min(NBUF, NCHUNK)):
        start_in(c, c % NBUF)

    for c in range(NCHUNK):
        p = c % NBUF
        wait_in(c, p)
        if c >= NBUF:
            out_copy(c - NBUF, p).wait()

        sub_buf = sub_bufs[p]
        subed_buf = subed_bufs[p]
        out_buf = out_bufs[p]

        def row_body(i, _):
            for f in range(FGROUPS):
                fs = pl.ds(f * LANES, LANES)
                x = subed_buf[i, fs]
                best = x - sub_buf[i + D, fs]
                besta = jnp.abs(best)
                for d in DELAYS[1:]:
                    r = x - sub_buf[i + D + d, fs]
                    ra = jnp.abs(r)
                    m = ra < besta
                    best = jnp.where(m, r, best)
                    besta = jnp.where(m, ra, besta)
                out_buf[i, fs] = best
            return 0

        lax.fori_loop(0, CH, row_body, 0)

        out_copy(c, p).start()
        if c + NBUF < NCHUNK:
            start_in(c + NBUF, p)

    for c in range(max(NCHUNK - NBUF, 0), NCHUNK):
        out_copy(c, c % NBUF).wait()


def _tc_body(subed_ref, sub_ref, out_ref):
    x = subed_ref[0]
    s = sub_ref[0]
    best = x - s
    besta = jnp.abs(best)
    for d in DELAYS[1:]:
        if d > 0:
            shifted = jnp.concatenate(
                [s[d:], jnp.broadcast_to(s[W - 1:], (d, F))], axis=0)
        else:
            shifted = jnp.concatenate(
                [jnp.broadcast_to(s[:1], (-d, F)), s[:W + d]], axis=0)
        r = x - shifted
        ra = jnp.abs(r)
        m = ra < besta
        best = jnp.where(m, r, best)
        besta = jnp.where(m, ra, besta)
    out_ref[0] = best


@jax.jit
def kernel(subed, sub):
    mesh = plsc.VectorSubcoreMesh(core_axis_name="c", subcore_axis_name="s")
    out_sc = pl.kernel(
        _sc_body,
        out_type=jax.ShapeDtypeStruct((NSC, W, F), jnp.float32),
        mesh=mesh,
        scratch_types=[
            [pltpu.VMEM((CH + 2 * D, F), jnp.float32) for _ in range(NBUF)],
            [pltpu.VMEM((CH, F), jnp.float32) for _ in range(NBUF)],
            [pltpu.VMEM((CH, F), jnp.float32) for _ in range(NBUF)],
            pltpu.SemaphoreType.DMA((NBUF, 2)),
            pltpu.SemaphoreType.DMA((NBUF,)),
        ],
        compiler_params=pltpu.CompilerParams(use_tc_tiling_on_sc=True),
    )(subed[:NSC], sub[:NSC])

    out_tc = pl.pallas_call(
        _tc_body,
        grid=(B - NSC,),
        in_specs=[
            pl.BlockSpec((1, W, F), lambda i: (i + NSC, 0, 0)),
            pl.BlockSpec((1, W, F), lambda i: (i + NSC, 0, 0)),
        ],
        out_specs=pl.BlockSpec((1, W, F), lambda i: (i + NSC, 0, 0)),
        out_shape=jax.ShapeDtypeStruct((B, W, F), jnp.float32),
    )(subed, sub)

    return lax.dynamic_update_slice(out_tc, out_sc, (0, 0, 0))


# speedup vs baseline: 1.1277x; 1.1277x over previous
import functools

import jax
import jax.numpy as jnp
from jax import lax
from jax.experimental import pallas as pl
from jax.experimental.pallas import tpu as pltpu
from jax.experimental.pallas import tpu_sc as plsc

W = 4096
F = 64
D = 8
K = 2 * D + 1
B = 8

NSC = 3

NUM_WORKERS = 32
ROWS_PER_WORKER = (NSC * W) // NUM_WORKERS
CH = 128
NCHUNK = ROWS_PER_WORKER // CH
NBUF = 2

DELAYS = [0] + [i for i in range(1, D + 1)] + [-i for i in range(1, D + 1)]

LANES = 16
FGROUPS = F // LANES


def _sc_body(subed_hbm, sub_hbm, out_hbm, sub_bufs, subed_bufs, out_bufs,
             sems_in, sems_out):
    wid = lax.axis_index("s") * 2 + lax.axis_index("c")
    row0 = wid * ROWS_PER_WORKER

    def chunk_coords(c):
        g = row0 + c * CH
        return g // W, g % W

    def sub_copy_mid(c, p):
        b, wloc = chunk_coords(c)
        return pltpu.make_async_copy(
            sub_hbm.at[b, pl.ds(wloc - D, CH + 2 * D)], sub_bufs[p],
            sems_in.at[p, 0])

    def sub_copy_first(c, p):
        b, _ = chunk_coords(c)
        return pltpu.make_async_copy(
            sub_hbm.at[b, pl.ds(0, CH + D)],
            sub_bufs[p].at[pl.ds(D, CH + D)], sems_in.at[p, 0])

    def sub_copy_last(c, p):
        b, wloc = chunk_coords(c)
        return pltpu.make_async_copy(
            sub_hbm.at[b, pl.ds(wloc - D, CH + D)],
            sub_bufs[p].at[pl.ds(0, CH + D)], sems_in.at[p, 0])

    def subed_copy(c, p):
        b, wloc = chunk_coords(c)
        return pltpu.make_async_copy(
            subed_hbm.at[b, pl.ds(wloc, CH)], subed_bufs[p],
            sems_in.at[p, 1])

    def edge_preds(c):
        _, wloc = chunk_coords(c)
        return wloc == 0, wloc == W - CH

    def start_in(c, p):
        subed_copy(c, p).start()
        is_first, is_last = edge_preds(c)

        @pl.when(is_first)
        def _():
            sub_copy_first(c, p).start()

        @pl.when(is_last)
        def _():
            sub_copy_last(c, p).start()

        @pl.when(jnp.logical_not(jnp.logical_or(is_first, is_last)))
        def _():
            sub_copy_mid(c, p).start()

    def wait_in(c, p):
        subed_copy(c, p).wait()
        is_first, is_last = edge_preds(c)

        @pl.when(is_first)
        def _():
            sub_copy_first(c, p).wait()
            for f in range(FGROUPS):
                fs = pl.ds(f * LANES, LANES)
                v = sub_bufs[p][D, fs]
                for t in range(D):
                    sub_bufs[p][t, fs] = v

        @pl.when(is_last)
        def _():
            sub_copy_last(c, p).wait()
            for f in range(FGROUPS):
                fs = pl.ds(f * LANES, LANES)
                v = sub_bufs[p][CH + D - 1, fs]
                for t in range(CH + D, CH + 2 * D):
                    sub_bufs[p][t, fs] = v

        @pl.when(jnp.logical_not(jnp.logical_or(is_first, is_last)))
        def _():
            sub_copy_mid(c, p).wait()

    def out_copy(c, p):
        b, wloc = chunk_coords(c)
        return pltpu.make_async_copy(
            out_bufs[p], out_hbm.at[b, pl.ds(wloc, CH)], sems_out.at[p])

    for c in range(min(NBUF, NCHUNK)):
        start_in(c, c % NBUF)

    for c in range(NCHUNK):
        p = c % NBUF
        wait_in(c, p)
        if c >= NBUF:
            out_copy(c - NBUF, p).wait()

        sub_buf = sub_bufs[p]
        subed_buf = subed_bufs[p]
        out_buf = out_bufs[p]

        def row_body(i, _):
            for f in range(FGROUPS):
                fs = pl.ds(f * LANES, LANES)
                x = subed_buf[i, fs]
                best = x - sub_buf[i + D, fs]
                besta = jnp.abs(best)
                for d in DELAYS[1:]:
                    r = x - sub_buf[i + D + d, fs]
                    ra = jnp.abs(r)
                    m = ra < besta
                    best = jnp.where(m, r, best)
                    besta = jnp.where(m, ra, besta)
                out_buf[i, fs] = best
            return 0

        lax.fori_loop(0, CH, row_body, 0)

        out_copy(c, p).start()
        if c + NBUF < NCHUNK:
            start_in(c + NBUF, p)

    for c in range(max(NCHUNK - NBUF, 0), NCHUNK):
        out_copy(c, c % NBUF).wait()


def _tc_body(subed_ref, sub_ref, out_ref):
    x = subed_ref[0]
    s = sub_ref[0]
    best = x - s
    besta = jnp.abs(best)
    for d in DELAYS[1:]:
        if d > 0:
            shifted = jnp.concatenate(
                [s[d:], jnp.broadcast_to(s[W - 1:], (d, F))], axis=0)
        else:
            shifted = jnp.concatenate(
                [jnp.broadcast_to(s[:1], (-d, F)), s[:W + d]], axis=0)
        r = x - shifted
        ra = jnp.abs(r)
        m = ra < besta
        best = jnp.where(m, r, best)
        besta = jnp.where(m, ra, besta)
    out_ref[0] = best


@jax.jit
def kernel(subed, sub):
    mesh = plsc.VectorSubcoreMesh(core_axis_name="c", subcore_axis_name="s")
    out_sc = pl.kernel(
        _sc_body,
        out_type=jax.ShapeDtypeStruct((NSC, W, F), jnp.float32),
        mesh=mesh,
        scratch_types=[
            [pltpu.VMEM((CH + 2 * D, F), jnp.float32) for _ in range(NBUF)],
            [pltpu.VMEM((CH, F), jnp.float32) for _ in range(NBUF)],
            [pltpu.VMEM((CH, F), jnp.float32) for _ in range(NBUF)],
            pltpu.SemaphoreType.DMA((NBUF, 2)),
            pltpu.SemaphoreType.DMA((NBUF,)),
        ],
        compiler_params=pltpu.CompilerParams(use_tc_tiling_on_sc=True),
    )(subed, sub)

    out_tc = pl.pallas_call(
        _tc_body,
        grid=(B - NSC,),
        in_specs=[
            pl.BlockSpec((1, W, F), lambda i: (i + NSC, 0, 0)),
            pl.BlockSpec((1, W, F), lambda i: (i + NSC, 0, 0)),
        ],
        out_specs=pl.BlockSpec((1, W, F), lambda i: (i + NSC, 0, 0)),
        out_shape=jax.ShapeDtypeStruct((B, W, F), jnp.float32),
    )(subed, sub)

    return lax.dynamic_update_slice(out_tc, out_sc, (0, 0, 0))
